# Initial kernel scaffold; baseline (speedup 1.0000x reference)
import jax
import jax.numpy as jnp
from jax.experimental import pallas as pl


def kernel(pooling_values, pooling_indices):
    raise NotImplementedError("write your pallas kernel here")


# speedup vs baseline: 16.2542x; 16.2542x over previous
import functools

import jax
import jax.numpy as jnp
from jax import lax
from jax.experimental import pallas as pl
from jax.experimental.pallas import tpu as pltpu
from jax.experimental.pallas import tpu_sc as plsc

POOL = 2
B, H, W, C = 2, 192, 192, 96
HW = H * W
M = (H * POOL) * (W * POOL)
G = 16
NSLAB = C // G
NHALF = 2
MH = M // NHALF
NROUND = NSLAB * NHALF
NS = 16
PPT = HW // NS
WINP = 384
NWIN = PPT // WINP
SROW = WINP * G // 128
ROWS_PT = MH // NS
ZCHUNK = 4608
NZ = MH * G // NS // ZCHUNK

_mesh = plsc.VectorSubcoreMesh(core_axis_name="c", subcore_axis_name="s")


WCH = 4608
NWCH = MH * G // NS // WCH


def _sc_body(vals_hbm, idx_hbm, out_hbm, accf, zeros_v, vals_w, idx_w,
             vals_s, tgt_s, wout1, wout2, sem_z, sem_in, sem_sc, sem_out):
  scid = lax.axis_index("c")
  sid = lax.axis_index("s")
  lane = lax.iota(jnp.int32, G)

  def fill_zero(i, carry):
    zeros_v[pl.ds(i * G, G)] = jnp.zeros((G,), jnp.float32)
    return carry

  lax.fori_loop(0, ZCHUNK // G, fill_zero, 0)

  def round_body(r, carry):
    slab = r % NSLAB
    half = r // NSLAB
    base = half * MH

    def zero(j, carry):
      pltpu.async_copy(
          zeros_v, accf.at[pl.ds(sid * (NZ * ZCHUNK) + j * ZCHUNK, ZCHUNK)],
          sem_z).wait()
      return carry

    lax.fori_loop(0, NZ, zero, 0)
    plsc.subcore_barrier()

    for wi in range(NWIN):
      p0 = sid * PPT + wi * WINP
      dv = pltpu.async_copy(
          vals_hbm.at[scid, pl.ds(p0, WINP), slab], vals_w, sem_in)
      di = pltpu.async_copy(
          idx_hbm.at[scid, pl.ds(p0, WINP), slab], idx_w, sem_in)
      dv.wait()
      di.wait()

      def compute(i, carry):
        for k in range(8):
          iv = idx_w[i * 8 + k, :]
          vv = vals_w[i * 8 + k, :]
          t = jnp.right_shift(iv, 5).astype(jnp.float32)
          o = ((t + 0.5) * (1.0 / 3.0)).astype(jnp.int32)
          ol = o - base
          ol2 = jnp.where(ol < 0, ol + MH, ol)
          fold = jnp.where(ol2 >= MH, ol2 - MH, ol2)
          vv1 = jnp.where(ol < 0, 0.0, vv)
          vm = jnp.where(ol >= MH, 0.0, vv1)
          tgt_s[i, pl.ds(k * G, G)] = fold * G + lane
          vals_s[i, pl.ds(k * G, G)] = vm
        return carry

      lax.fori_loop(0, SROW, compute, 0)

      def scat(j, carry):
        pltpu.async_copy(
            vals_s.at[j], accf.at[tgt_s.at[j]], sem_sc, add=True).wait()
        return carry

      lax.fori_loop(0, SROW, scat, 0)

    plsc.subcore_barrier()

    def wout(c, carry):
      w0 = sid * (MH * G // NS) + c * WCH
      pltpu.sync_copy(accf.at[pl.ds(w0, WCH)], wout1)

      def repack(i, carry2):
        wout2[i, :] = wout1[pl.ds(i * G, G)]
        return carry2

      lax.fori_loop(0, WCH // G, repack, 0)
      o0 = sid * ROWS_PT + c * (WCH // G)
      pltpu.sync_copy(
          wout2, out_hbm.at[scid, half, pl.ds(o0, WCH // G), slab])
      return carry

    lax.fori_loop(0, NWCH, wout, 0)
    return carry

  lax.fori_loop(0, NROUND, round_body, 0)


_unpool = functools.partial(
    pl.kernel,
    out_type=jax.ShapeDtypeStruct((B, NHALF, MH, NSLAB, G), jnp.float32),
    mesh=_mesh,
    compiler_params=pltpu.CompilerParams(use_tc_tiling_on_sc=False),
    scratch_types=[
        pltpu.VMEM_SHARED((MH * G,), jnp.float32),
        pltpu.VMEM((ZCHUNK,), jnp.float32),
        pltpu.VMEM((WINP, G), jnp.float32),
        pltpu.VMEM((WINP, G), jnp.int32),
        pltpu.VMEM((SROW, 128), jnp.float32),
        pltpu.VMEM((SROW, 128), jnp.int32),
        pltpu.VMEM((WCH,), jnp.float32),
        pltpu.VMEM((WCH // G, G), jnp.float32),
        pltpu.SemaphoreType.DMA,
        pltpu.SemaphoreType.DMA,
        pltpu.SemaphoreType.DMA,
        pltpu.SemaphoreType.DMA,
    ],
)(_sc_body)


@jax.jit
def kernel(pooling_values, pooling_indices):
  vals4 = pooling_values.reshape(B, HW, NSLAB, G)
  idx4 = pooling_indices.astype(jnp.int32).reshape(B, HW, NSLAB, G)
  out5 = _unpool(vals4, idx4)
  return out5.reshape(B, H * POOL, W * POOL, C)


# speedup vs baseline: 22.7997x; 1.4027x over previous
import functools

import jax
import jax.numpy as jnp
from jax import lax
from jax.experimental import pallas as pl
from jax.experimental.pallas import tpu as pltpu
from jax.experimental.pallas import tpu_sc as plsc

POOL = 2
B, H, W, C = 2, 192, 192, 96
HW = H * W
M = (H * POOL) * (W * POOL)
G = 16
NSLAB = C // G
NHALF = 2
MH = M // NHALF
NROUND = NSLAB * NHALF
NS = 16
PPT = HW // NS
WINP = 384
NWIN = PPT // WINP
SROW = WINP * G // 128
ROWS_PT = MH // NS
WPT = MH * G // NS
WCH = 2304
NWCH = WPT // WCH

_mesh = plsc.VectorSubcoreMesh(core_axis_name="c", subcore_axis_name="s")


def _sc_body(vals_hbm, idx_hbm, out_hbm, accf, zeros_v,
             vals_w0, idx_w0, vals_w1, idx_w1, vals_s, tgt_s,
             wout1a, wout2a, wout1b, wout2b,
             sem_z, sem_in, sem_sc, sem_out):
  scid = lax.axis_index("c")
  sid = lax.axis_index("s")
  lane = lax.iota(jnp.int32, G)

  def fill_zero(i, carry):
    zeros_v[pl.ds(i * G, G)] = jnp.zeros((G,), jnp.float32)
    return carry

  lax.fori_loop(0, WCH // G, fill_zero, 0)

  zd = [
      pltpu.async_copy(zeros_v, accf.at[pl.ds(sid * WPT + j * WCH, WCH)],
                       sem_z)
      for j in range(NWCH)
  ]
  for d in zd:
    d.wait()

  win_bufs = [(vals_w0, idx_w0), (vals_w1, idx_w1)]
  w1b = [wout1a, wout1b]
  w2b = [wout2a, wout2b]

  def round_body(r, carry):
    slab = r % NSLAB
    half = r // NSLAB
    base = half * MH

    plsc.subcore_barrier()

    def win_in(wi, bufs):
      p0 = sid * PPT + wi * WINP
      return (pltpu.async_copy(
          vals_hbm.at[scid, pl.ds(p0, WINP), slab], bufs[0], sem_in),
              pltpu.async_copy(
          idx_hbm.at[scid, pl.ds(p0, WINP), slab], bufs[1], sem_in))

    pending = {0: win_in(0, win_bufs[0])}
    for wi in range(NWIN):
      vw, iw = win_bufs[wi % 2]
      if wi + 1 < NWIN:
        pending[wi + 1] = win_in(wi + 1, win_bufs[(wi + 1) % 2])
      da, db = pending.pop(wi)
      da.wait()
      db.wait()

      def compute(i, carry2, vw=vw, iw=iw, base=base):
        for k in range(8):
          iv = iw[i * 8 + k, :]
          vv = vw[i * 8 + k, :]
          t = jnp.right_shift(iv, 5).astype(jnp.float32)
          o = ((t + 0.5) * (1.0 / 3.0)).astype(jnp.int32)
          ol = o - base
          ol2 = jnp.where(ol < 0, ol + MH, ol)
          fold = jnp.where(ol2 >= MH, ol2 - MH, ol2)
          vv1 = jnp.where(ol < 0, 0.0, vv)
          vm = jnp.where(ol >= MH, 0.0, vv1)
          tgt_s[i, pl.ds(k * G, G)] = fold * G + lane
          vals_s[i, pl.ds(k * G, G)] = vm
        return carry2

      lax.fori_loop(0, SROW, compute, 0)

      def scat(j, carry2):
        ds_ = [
            pltpu.async_copy(
                vals_s.at[j * 8 + k], accf.at[tgt_s.at[j * 8 + k]],
                sem_sc, add=True)
            for k in range(8)
        ]
        for d in ds_:
          d.wait()
        return carry2

      lax.fori_loop(0, SROW // 8, scat, 0)

    plsc.subcore_barrier()

    zlist = []
    olist = [None, None]
    for c in range(NWCH):
      cur = c % 2
      if olist[cur] is not None:
        olist[cur].wait()
      w0 = sid * WPT + c * WCH
      pltpu.sync_copy(accf.at[pl.ds(w0, WCH)], w1b[cur])
      zlist.append(
          pltpu.async_copy(zeros_v, accf.at[pl.ds(w0, WCH)], sem_z))

      def repack(i, carry2, cur=cur):
        w2b[cur][i, :] = w1b[cur][pl.ds(i * G, G)]
        return carry2

      lax.fori_loop(0, WCH // G, repack, 0)
      o0 = sid * ROWS_PT + c * (WCH // G)
      olist[cur] = pltpu.async_copy(
          w2b[cur], out_hbm.at[scid, half, pl.ds(o0, WCH // G), slab],
          sem_out)
    olist[0].wait()
    olist[1].wait()
    for d in zlist:
      d.wait()
    return carry

  lax.fori_loop(0, NROUND, round_body, 0)


_unpool = functools.partial(
    pl.kernel,
    out_type=jax.ShapeDtypeStruct((B, NHALF, MH, NSLAB, G), jnp.float32),
    mesh=_mesh,
    compiler_params=pltpu.CompilerParams(use_tc_tiling_on_sc=False),
    scratch_types=[
        pltpu.VMEM_SHARED((MH * G,), jnp.float32),
        pltpu.VMEM((WCH,), jnp.float32),
        pltpu.VMEM((WINP, G), jnp.float32),
        pltpu.VMEM((WINP, G), jnp.int32),
        pltpu.VMEM((WINP, G), jnp.float32),
        pltpu.VMEM((WINP, G), jnp.int32),
        pltpu.VMEM((SROW, 128), jnp.float32),
        pltpu.VMEM((SROW, 128), jnp.int32),
        pltpu.VMEM((WCH,), jnp.float32),
        pltpu.VMEM((WCH // G, G), jnp.float32),
        pltpu.VMEM((WCH,), jnp.float32),
        pltpu.VMEM((WCH // G, G), jnp.float32),
        pltpu.SemaphoreType.DMA,
        pltpu.SemaphoreType.DMA,
        pltpu.SemaphoreType.DMA,
        pltpu.SemaphoreType.DMA,
    ],
)(_sc_body)


@jax.jit
def kernel(pooling_values, pooling_indices):
  vals4 = pooling_values.reshape(B, HW, NSLAB, G)
  idx4 = pooling_indices.astype(jnp.int32).reshape(B, HW, NSLAB, G)
  out5 = _unpool(vals4, idx4)
  return out5.reshape(B, H * POOL, W * POOL, C)


# speedup vs baseline: 23.5070x; 1.0310x over previous
import functools

import jax
import jax.numpy as jnp
from jax import lax
from jax.experimental import pallas as pl
from jax.experimental.pallas import tpu as pltpu
from jax.experimental.pallas import tpu_sc as plsc

POOL = 2
B, H, W, C = 2, 192, 192, 96
HW = H * W
M = (H * POOL) * (W * POOL)
G = 16
NSLAB = C // G
NHALF = 2
MH = M // NHALF
NROUND = NSLAB * NHALF
NS = 16
PPT = HW // NS
WINP = 384
NWIN = PPT // WINP
SROW = WINP * G // 128
ROWS_PT = MH // NS
WPT = MH * G // NS
WCH = 2304
NWCH = WPT // WCH

_mesh = plsc.VectorSubcoreMesh(core_axis_name="c", subcore_axis_name="s")


def _sc_body(vals_hbm, idx_hbm, out_hbm, accf, zeros_v,
             vals_w0, idx_w0, vals_w1, idx_w1, vals_s, tgt_s,
             wout1a, wout2a, wout1b, wout2b,
             sem_z, sem_in, sem_sc, sem_out):
  scid = lax.axis_index("c")
  sid = lax.axis_index("s")
  lane = lax.iota(jnp.int32, G)

  def fill_zero(i, carry):
    zeros_v[pl.ds(i * G, G)] = jnp.zeros((G,), jnp.float32)
    return carry

  lax.fori_loop(0, WCH // G, fill_zero, 0)

  zd = [
      pltpu.async_copy(zeros_v, accf.at[pl.ds(sid * WPT + j * WCH, WCH)],
                       sem_z)
      for j in range(NWCH)
  ]
  for d in zd:
    d.wait()

  win_bufs = [(vals_w0, idx_w0), (vals_w1, idx_w1)]
  w1b = [wout1a, wout1b]
  w2b = [wout2a, wout2b]

  def round_body(r, carry):
    slab = r % NSLAB
    half = r // NSLAB
    base = half * MH

    plsc.subcore_barrier()

    def win_in(wi, bufs):
      p0 = sid * PPT + wi * WINP
      return (pltpu.async_copy(
          vals_hbm.at[scid, pl.ds(p0, WINP), slab], bufs[0], sem_in),
              pltpu.async_copy(
          idx_hbm.at[scid, pl.ds(p0, WINP), slab], bufs[1], sem_in))

    pending = {0: win_in(0, win_bufs[0])}
    for wi in range(NWIN):
      vw, iw = win_bufs[wi % 2]
      if wi + 1 < NWIN:
        pending[wi + 1] = win_in(wi + 1, win_bufs[(wi + 1) % 2])
      da, db = pending.pop(wi)
      da.wait()
      db.wait()

      def compute(i, carry2, vw=vw, iw=iw, base=base):
        for k in range(8):
          iv = iw[i * 8 + k, :]
          vv = vw[i * 8 + k, :]
          t = jnp.right_shift(iv, 5).astype(jnp.float32)
          o = ((t + 0.5) * (1.0 / 3.0)).astype(jnp.int32)
          ol = o - base
          inr = plsc.bitcast(ol, jnp.uint32) < jnp.uint32(MH)
          tgt_s[i, pl.ds(k * G, G)] = jnp.where(inr, ol * G + lane, -1)
          vals_s[i, pl.ds(k * G, G)] = vv
        return carry2

      lax.fori_loop(0, SROW, compute, 0)

      def scat(j, carry2):
        ds_ = [
            pltpu.async_copy(
                vals_s.at[j * 8 + k],
                accf.at[plsc.Indices(tgt_s.at[j * 8 + k], ignored_value=-1)],
                sem_sc, add=True)
            for k in range(8)
        ]
        for d in ds_:
          d.wait()
        return carry2

      lax.fori_loop(0, SROW // 8, scat, 0)

    plsc.subcore_barrier()

    zlist = []
    olist = [None, None]
    for c in range(NWCH):
      cur = c % 2
      if olist[cur] is not None:
        olist[cur].wait()
      w0 = sid * WPT + c * WCH
      pltpu.sync_copy(accf.at[pl.ds(w0, WCH)], w1b[cur])
      zlist.append(
          pltpu.async_copy(zeros_v, accf.at[pl.ds(w0, WCH)], sem_z))

      def repack(i, carry2, cur=cur):
        w2b[cur][i, :] = w1b[cur][pl.ds(i * G, G)]
        return carry2

      lax.fori_loop(0, WCH // G, repack, 0)
      o0 = sid * ROWS_PT + c * (WCH // G)
      olist[cur] = pltpu.async_copy(
          w2b[cur], out_hbm.at[scid, half, pl.ds(o0, WCH // G), slab],
          sem_out)
    olist[0].wait()
    olist[1].wait()
    for d in zlist:
      d.wait()
    return carry

  lax.fori_loop(0, NROUND, round_body, 0)


_unpool = functools.partial(
    pl.kernel,
    out_type=jax.ShapeDtypeStruct((B, NHALF, MH, NSLAB, G), jnp.float32),
    mesh=_mesh,
    compiler_params=pltpu.CompilerParams(use_tc_tiling_on_sc=False),
    scratch_types=[
        pltpu.VMEM_SHARED((MH * G,), jnp.float32),
        pltpu.VMEM((WCH,), jnp.float32),
        pltpu.VMEM((WINP, G), jnp.float32),
        pltpu.VMEM((WINP, G), jnp.int32),
        pltpu.VMEM((WINP, G), jnp.float32),
        pltpu.VMEM((WINP, G), jnp.int32),
        pltpu.VMEM((SROW, 128), jnp.float32),
        pltpu.VMEM((SROW, 128), jnp.int32),
        pltpu.VMEM((WCH,), jnp.float32),
        pltpu.VMEM((WCH // G, G), jnp.float32),
        pltpu.VMEM((WCH,), jnp.float32),
        pltpu.VMEM((WCH // G, G), jnp.float32),
        pltpu.SemaphoreType.DMA,
        pltpu.SemaphoreType.DMA,
        pltpu.SemaphoreType.DMA,
        pltpu.SemaphoreType.DMA,
    ],
)(_sc_body)


@jax.jit
def kernel(pooling_values, pooling_indices):
  vals4 = pooling_values.reshape(B, HW, NSLAB, G)
  idx4 = pooling_indices.astype(jnp.int32).reshape(B, HW, NSLAB, G)
  out5 = _unpool(vals4, idx4)
  return out5.reshape(B, H * POOL, W * POOL, C)


# speedup vs baseline: 31.6744x; 1.3474x over previous
import functools

import jax
import jax.numpy as jnp
from jax import lax
from jax.experimental import pallas as pl
from jax.experimental.pallas import tpu as pltpu
from jax.experimental.pallas import tpu_sc as plsc

POOL = 2
B, H, W, C = 2, 192, 192, 96
HW = H * W
M = (H * POOL) * (W * POOL)
G = 16
NSLAB = C // G
NHALF = 2
MH = M // NHALF
NROUND = NSLAB * NHALF
NS = 16
PPT = HW // NS
WINP = 384
NWIN = PPT // WINP
SROW = WINP * G // 128
ROWS_PT = MH // NS
WPT = MH * G // NS
WCH = 2304
NWCH = WPT // WCH

_mesh = plsc.VectorSubcoreMesh(core_axis_name="c", subcore_axis_name="s")


def _sc_body(vals_hbm, idx_hbm, out_hbm, accf, zeros_v,
             vals_w0, idx_w0, vals_w1, idx_w1, vals_s, tgt_s,
             wout1a, wout2a, wout1b, wout2b,
             sem_z, sem_in, sem_sc, sem_out, sem_wi):
  scid = lax.axis_index("c")
  sid = lax.axis_index("s")
  lane = lax.iota(jnp.int32, G)

  def fill_zero(i, carry):
    zeros_v[pl.ds(i * G, G)] = jnp.zeros((G,), jnp.float32)
    return carry

  lax.fori_loop(0, WCH // G, fill_zero, 0)

  zd = [
      pltpu.async_copy(zeros_v, accf.at[pl.ds(sid * WPT + j * WCH, WCH)],
                       sem_z)
      for j in range(NWCH)
  ]
  for d in zd:
    d.wait()

  win_bufs = [(vals_w0, idx_w0), (vals_w1, idx_w1)]
  w1b = [wout1a, wout1b]
  w2b = [wout2a, wout2b]

  def round_body(r, carry):
    slab = r % NSLAB
    half = r // NSLAB
    base = half * MH

    plsc.subcore_barrier()

    def win_in(wi, bufs):
      p0 = sid * PPT + wi * WINP
      return (pltpu.async_copy(
          vals_hbm.at[scid, pl.ds(p0, WINP), slab], bufs[0], sem_in),
              pltpu.async_copy(
          idx_hbm.at[scid, pl.ds(p0, WINP), slab], bufs[1], sem_in))

    pending = {0: win_in(0, win_bufs[0])}
    for wi in range(NWIN):
      vw, iw = win_bufs[wi % 2]
      if wi + 1 < NWIN:
        pending[wi + 1] = win_in(wi + 1, win_bufs[(wi + 1) % 2])
      da, db = pending.pop(wi)
      da.wait()
      db.wait()

      def compute(i, carry2, vw=vw, iw=iw, base=base):
        for k in range(8):
          iv = iw[i * 8 + k, :]
          vv = vw[i * 8 + k, :]
          t = jnp.right_shift(iv, 5).astype(jnp.float32)
          o = ((t + 0.5) * (1.0 / 3.0)).astype(jnp.int32)
          ol = o - base
          inr = plsc.bitcast(ol, jnp.uint32) < jnp.uint32(MH)
          tgt_s[i, pl.ds(k * G, G)] = jnp.where(inr, ol * G + lane, -1)
          vals_s[i, pl.ds(k * G, G)] = vv
        return carry2

      lax.fori_loop(0, SROW, compute, 0)

      batches = []
      for j in range(SROW // 8):
        batches.append([
            pltpu.async_copy(
                vals_s.at[j * 8 + k],
                accf.at[plsc.Indices(tgt_s.at[j * 8 + k], ignored_value=-1)],
                sem_sc, add=True)
            for k in range(8)
        ])
        if j >= 1:
          for d in batches[j - 1]:
            d.wait()
      for d in batches[-1]:
        d.wait()

    plsc.subcore_barrier()

    zlist = []
    olist = [None, None]
    din = [None, None]
    din[0] = pltpu.async_copy(
        accf.at[pl.ds(sid * WPT, WCH)], w1b[0], sem_wi)
    for c in range(NWCH):
      cur = c % 2
      if c + 1 < NWCH:
        din[1 - cur] = pltpu.async_copy(
            accf.at[pl.ds(sid * WPT + (c + 1) * WCH, WCH)], w1b[1 - cur],
            sem_wi)
      din[cur].wait()
      zlist.append(pltpu.async_copy(
          zeros_v, accf.at[pl.ds(sid * WPT + c * WCH, WCH)], sem_z))
      if olist[cur] is not None:
        olist[cur].wait()

      def repack(i, carry2, cur=cur):
        for k in range(8):
          w2b[cur][i * 8 + k, :] = w1b[cur][pl.ds((i * 8 + k) * G, G)]
        return carry2

      lax.fori_loop(0, WCH // G // 8, repack, 0)
      o0 = sid * ROWS_PT + c * (WCH // G)
      olist[cur] = pltpu.async_copy(
          w2b[cur], out_hbm.at[scid, half, pl.ds(o0, WCH // G), slab],
          sem_out)
    olist[0].wait()
    olist[1].wait()
    for d in zlist:
      d.wait()
    return carry

  lax.fori_loop(0, NROUND, round_body, 0)


_unpool = functools.partial(
    pl.kernel,
    out_type=jax.ShapeDtypeStruct((B, NHALF, MH, NSLAB, G), jnp.float32),
    mesh=_mesh,
    compiler_params=pltpu.CompilerParams(use_tc_tiling_on_sc=False),
    scratch_types=[
        pltpu.VMEM_SHARED((MH * G,), jnp.float32),
        pltpu.VMEM((WCH,), jnp.float32),
        pltpu.VMEM((WINP, G), jnp.float32),
        pltpu.VMEM((WINP, G), jnp.int32),
        pltpu.VMEM((WINP, G), jnp.float32),
        pltpu.VMEM((WINP, G), jnp.int32),
        pltpu.VMEM((SROW, 128), jnp.float32),
        pltpu.VMEM((SROW, 128), jnp.int32),
        pltpu.VMEM((WCH,), jnp.float32),
        pltpu.VMEM((WCH // G, G), jnp.float32),
        pltpu.VMEM((WCH,), jnp.float32),
        pltpu.VMEM((WCH // G, G), jnp.float32),
        pltpu.SemaphoreType.DMA,
        pltpu.SemaphoreType.DMA,
        pltpu.SemaphoreType.DMA,
        pltpu.SemaphoreType.DMA,
        pltpu.SemaphoreType.DMA,
    ],
)(_sc_body)


@jax.jit
def kernel(pooling_values, pooling_indices):
  vals4 = pooling_values.reshape(B, HW, NSLAB, G)
  idx4 = pooling_indices.astype(jnp.int32).reshape(B, HW, NSLAB, G)
  out5 = _unpool(vals4, idx4)
  return out5.reshape(B, H * POOL, W * POOL, C)
